# baseline (speedup 1.0000x reference)
import jax
import jax.numpy as jnp
from jax import lax
from jax.experimental import pallas as pl
from jax.experimental.pallas import tpu as pltpu
from jax.experimental.pallas import tpu_sc as plsc

DIM = 64
NC = 2
NS = 16
NW = NC * NS

CHUNK = 1600


def _emb_gather(x_hbm, table_hbm, out_hbm, idx_v, rows_v, sem):
    b_per_w = out_hbm.shape[0] // NW
    n_chunks = b_per_w // CHUNK
    wid = lax.axis_index("s") * NC + lax.axis_index("c")
    base = wid * b_per_w

    def body(i, carry):
        off = pl.multiple_of(base + i * CHUNK, CHUNK)
        pltpu.sync_copy(x_hbm.at[pl.ds(off, CHUNK)], idx_v)
        pltpu.async_copy(table_hbm.at[idx_v], rows_v, sem).wait()
        pltpu.sync_copy(rows_v, out_hbm.at[pl.ds(off, CHUNK)])
        return carry

    lax.fori_loop(0, n_chunks, body, 0)


def kernel(x, table):
    b, s = x.shape
    total = b * s
    xf = x.reshape(total)
    out = pl.kernel(
        _emb_gather,
        out_type=jax.ShapeDtypeStruct((total, DIM), jnp.float32),
        mesh=plsc.VectorSubcoreMesh(core_axis_name="c", subcore_axis_name="s"),
        compiler_params=pltpu.CompilerParams(use_tc_tiling_on_sc=False),
        scratch_types=[
            pltpu.VMEM((CHUNK,), jnp.int32),
            pltpu.VMEM((CHUNK, DIM), jnp.float32),
            pltpu.SemaphoreType.DMA,
        ],
    )(xf, table)
    return out.reshape(b, s, DIM)


# speedup vs baseline: 1.0036x; 1.0036x over previous
import jax
import jax.numpy as jnp
from jax import lax
from jax.experimental import pallas as pl
from jax.experimental.pallas import tpu as pltpu
from jax.experimental.pallas import tpu_sc as plsc

DIM = 64
NC = 2
NS = 16
NW = NC * NS

CHUNK = 800


def _emb_gather(x_hbm, table_hbm, out_hbm, idx_v, rows0, rows1, sem0, sem1):
    b_per_w = out_hbm.shape[0] // NW
    n_chunks = b_per_w // CHUNK
    wid = lax.axis_index("s") * NC + lax.axis_index("c")
    base = wid * b_per_w

    pltpu.sync_copy(x_hbm.at[pl.ds(base, b_per_w)], idx_v)

    def gather(c, buf, sem):
        off = pl.multiple_of(c * CHUNK, CHUNK)
        return pltpu.async_copy(table_hbm.at[idx_v.at[pl.ds(off, CHUNK)]], buf, sem)

    def store(c, buf):
        off = pl.multiple_of(base + c * CHUNK, CHUNK)
        pltpu.sync_copy(buf, out_hbm.at[pl.ds(off, CHUNK)])

    gather(0, rows0, sem0)

    def body(j, carry):
        c0 = 2 * j
        c1 = c0 + 1
        pltpu.make_async_copy(table_hbm.at[idx_v.at[pl.ds(0, CHUNK)]], rows0, sem0).wait()
        gather(c1, rows1, sem1)
        store(c0, rows0)
        pltpu.make_async_copy(table_hbm.at[idx_v.at[pl.ds(0, CHUNK)]], rows1, sem1).wait()

        @pl.when(c1 + 1 < n_chunks)
        def _():
            gather(c1 + 1, rows0, sem0)

        store(c1, rows1)
        return carry

    lax.fori_loop(0, n_chunks // 2, body, 0)


def kernel(x, table):
    b, s = x.shape
    total = b * s
    b_per_w = total // NW
    xf = x.reshape(total)
    out = pl.kernel(
        _emb_gather,
        out_type=jax.ShapeDtypeStruct((total, DIM), jnp.float32),
        mesh=plsc.VectorSubcoreMesh(core_axis_name="c", subcore_axis_name="s"),
        compiler_params=pltpu.CompilerParams(use_tc_tiling_on_sc=False),
        scratch_types=[
            pltpu.VMEM((b_per_w,), jnp.int32),
            pltpu.VMEM((CHUNK, DIM), jnp.float32),
            pltpu.VMEM((CHUNK, DIM), jnp.float32),
            pltpu.SemaphoreType.DMA,
            pltpu.SemaphoreType.DMA,
        ],
    )(xf, table)
    return out.reshape(b, s, DIM)
